# Initial kernel scaffold; baseline (speedup 1.0000x reference)
import jax
import jax.numpy as jnp
from jax.experimental import pallas as pl


def kernel(h, p, e, edge_index, snorm_n, Qh_W, Kh_W, Vh_W, Eh_W, Qp_W, Kp_W, Ep_W, Vp_W, Oh_W, Oh_b, Op_W, Op_b, FFN1_W, FFN1_b, FFN2_W, FFN2_b, bn1_g, bn1_b, bn2_g, bn2_b):
    raise NotImplementedError("write your pallas kernel here")


# speedup vs baseline: 1.0931x; 1.0931x over previous
import functools

import jax
import jax.numpy as jnp
import numpy as np
from jax.experimental import pallas as pl
from jax.experimental.pallas import tpu as pltpu

N = 10000
E = 320000
D = 128
H = 8
DH = D // H


def _pre_body(h, p, qhw_h, qhw_p, khw_h, khw_p, vhw_h, vhw_p,
              qpw, kpw, vpw,
              qh_o, kh_o, vh_o, qp_o, kp_o, vp_o):
    hv = h[...]
    pv = p[...]
    scale = 1.0 / np.sqrt(DH)
    qh_o[...] = (jnp.dot(hv, qhw_h[...], preferred_element_type=jnp.float32)
                 + jnp.dot(pv, qhw_p[...], preferred_element_type=jnp.float32)) * scale
    kh_o[...] = (jnp.dot(hv, khw_h[...], preferred_element_type=jnp.float32)
                 + jnp.dot(pv, khw_p[...], preferred_element_type=jnp.float32))
    vh_o[...] = (jnp.dot(hv, vhw_h[...], preferred_element_type=jnp.float32)
                 + jnp.dot(pv, vhw_p[...], preferred_element_type=jnp.float32))
    qp_o[...] = jnp.dot(pv, qpw[...], preferred_element_type=jnp.float32) * scale
    kp_o[...] = jnp.dot(pv, kpw[...], preferred_element_type=jnp.float32)
    vp_o[...] = jnp.dot(pv, vpw[...], preferred_element_type=jnp.float32)


def _pre(h, p, Qh_W, Kh_W, Vh_W, Qp_W, Kp_W, Vp_W):
    outs = [jax.ShapeDtypeStruct((N, D), jnp.float32)] * 6
    return pl.pallas_call(
        _pre_body,
        out_shape=outs,
    )(h, p, Qh_W[:D], Qh_W[D:], Kh_W[:D], Kh_W[D:], Vh_W[:D], Vh_W[D:],
      Qp_W, Kp_W, Vp_W)


def _post_body(wvh, zh, wvp, zp, h, p, ohw, ohb, opw, opb,
               f1w, f1b, f2w, f2b, g1, b1, g2, b2, hh_o, pp_o):
    head_of_col = jax.lax.broadcasted_iota(jnp.int32, (H, D), 1) // DH
    row = jax.lax.broadcasted_iota(jnp.int32, (H, D), 0)
    sel = (head_of_col == row).astype(jnp.float32)
    zhf = jnp.dot(zh[...], sel, preferred_element_type=jnp.float32)
    zpf = jnp.dot(zp[...], sel, preferred_element_type=jnp.float32)
    h_attn = wvh[...] / (zhf + 1e-6)
    p_attn = wvp[...] / (zpf + 1e-6)
    hh = jnp.dot(h_attn, ohw[...], preferred_element_type=jnp.float32) + ohb[...]
    pp = jnp.tanh(jnp.dot(p_attn, opw[...], preferred_element_type=jnp.float32) + opb[...])
    hh = h[...] + hh
    pp_o[...] = p[...] + pp
    mu = jnp.mean(hh, axis=0, keepdims=True)
    var = jnp.mean((hh - mu) ** 2, axis=0, keepdims=True)
    hh = g1[...] * (hh - mu) / jnp.sqrt(var + 1e-5) + b1[...]
    h_in2 = hh
    hh = jnp.maximum(jnp.dot(hh, f1w[...], preferred_element_type=jnp.float32) + f1b[...], 0.0)
    hh = jnp.dot(hh, f2w[...], preferred_element_type=jnp.float32) + f2b[...]
    hh = h_in2 + hh
    mu2 = jnp.mean(hh, axis=0, keepdims=True)
    var2 = jnp.mean((hh - mu2) ** 2, axis=0, keepdims=True)
    hh_o[...] = g2[...] * (hh - mu2) / jnp.sqrt(var2 + 1e-5) + b2[...]


def _post(wvh, zh, wvp, zp, h, p, Oh_W, Oh_b, Op_W, Op_b,
          FFN1_W, FFN1_b, FFN2_W, FFN2_b, bn1_g, bn1_b, bn2_g, bn2_b):
    outs = [jax.ShapeDtypeStruct((N, D), jnp.float32)] * 2
    return pl.pallas_call(
        _post_body,
        out_shape=outs,
    )(wvh, zh, wvp, zp, h, p, Oh_W, Oh_b.reshape(1, D), Op_W, Op_b.reshape(1, D),
      FFN1_W, FFN1_b.reshape(1, 2 * D), FFN2_W, FFN2_b.reshape(1, D),
      bn1_g.reshape(1, D), bn1_b.reshape(1, D), bn2_g.reshape(1, D), bn2_b.reshape(1, D))


def kernel(h, p, e, edge_index, snorm_n, Qh_W, Kh_W, Vh_W, Eh_W, Qp_W, Kp_W,
           Ep_W, Vp_W, Oh_W, Oh_b, Op_W, Op_b, FFN1_W, FFN1_b, FFN2_W,
           FFN2_b, bn1_g, bn1_b, bn2_g, bn2_b):
    src = edge_index[0]
    dst = edge_index[1]
    qh, kh, vh, qp, kp, vp = _pre(h, p, Qh_W, Kh_W, Vh_W, Qp_W, Kp_W, Vp_W)

    def edge_stage(q, k, v, ew):
        ee = (e @ ew).reshape(E, H, DH)
        score = (k[src].reshape(E, H, DH) * q[dst].reshape(E, H, DH)) * ee
        s = jnp.exp(jnp.clip(jnp.sum(score, axis=-1), -5.0, 5.0))
        wv = jax.ops.segment_sum(
            v[src].reshape(E, H, DH) * s[..., None], dst, num_segments=N
        ).reshape(N, D)
        z = jax.ops.segment_sum(s, dst, num_segments=N)
        return wv, z

    wvh, zh = edge_stage(qh, kh, vh, Eh_W)
    wvp, zp = edge_stage(qp, kp, vp, Ep_W)

    return _post(wvh, zh, wvp, zp, h, p, Oh_W, Oh_b, Op_W, Op_b,
                 FFN1_W, FFN1_b, FFN2_W, FFN2_b, bn1_g, bn1_b, bn2_g, bn2_b)


# speedup vs baseline: 34.3680x; 31.4407x over previous
import functools

import jax
import jax.numpy as jnp
import numpy as np
from jax import lax
from jax.experimental import pallas as pl
from jax.experimental.pallas import tpu as pltpu
from jax.experimental.pallas import tpu_sc as plsc

N = 10000
E = 320000
D = 128
H = 8
DH = D // H

_B = 32
_EPT = E // 16
_NBLK = _EPT // _B
_ROWS = 664


def _permute16(x, idx):
    dn = lax.GatherDimensionNumbers(offset_dims=(), collapsed_slice_dims=(0,),
                                    start_index_map=(0,))
    return lax.gather(x, idx[:, None], dn, slice_sizes=(1,),
                      mode=lax.GatherScatterMode.PROMISE_IN_BOUNDS)


def _pre_body(h, p, qhw_h, qhw_p, khw_h, khw_p, vhw_h, vhw_p,
              qpw, kpw, vpw,
              qh_o, kh_o, vh_o, qp_o, kp_o, vp_o):
    hv = h[...]
    pv = p[...]
    scale = 1.0 / np.sqrt(DH)
    qh_o[...] = (jnp.dot(hv, qhw_h[...], preferred_element_type=jnp.float32)
                 + jnp.dot(pv, qhw_p[...], preferred_element_type=jnp.float32)) * scale
    kh_o[...] = (jnp.dot(hv, khw_h[...], preferred_element_type=jnp.float32)
                 + jnp.dot(pv, khw_p[...], preferred_element_type=jnp.float32))
    vh_o[...] = (jnp.dot(hv, vhw_h[...], preferred_element_type=jnp.float32)
                 + jnp.dot(pv, vhw_p[...], preferred_element_type=jnp.float32))
    qp_o[...] = jnp.dot(pv, qpw[...], preferred_element_type=jnp.float32) * scale
    kp_o[...] = jnp.dot(pv, kpw[...], preferred_element_type=jnp.float32)
    vp_o[...] = jnp.dot(pv, vpw[...], preferred_element_type=jnp.float32)


def _pre(h, p, Qh_W, Kh_W, Vh_W, Qp_W, Kp_W, Vp_W):
    outs = [jax.ShapeDtypeStruct((N, D), jnp.float32)] * 6
    return pl.pallas_call(
        _pre_body,
        out_shape=outs,
    )(h, p, Qh_W[:D], Qh_W[D:], Kh_W[:D], Kh_W[D:], Vh_W[:D], Vh_W[D:],
      Qp_W, Kp_W, Vp_W)


def _eproj_body(e_blk, ehw, epw, eh_o, ep_o):
    ev = e_blk[...]
    eh_o[...] = jnp.dot(ev, ehw[...], preferred_element_type=jnp.float32)
    ep_o[...] = jnp.dot(ev, epw[...], preferred_element_type=jnp.float32)


def _eproj(e, Eh_W, Ep_W):
    BLK = 2000
    return pl.pallas_call(
        _eproj_body,
        grid=(E // BLK,),
        in_specs=[pl.BlockSpec((BLK, D), lambda i: (i, 0)),
                  pl.BlockSpec((D, D), lambda i: (0, 0)),
                  pl.BlockSpec((D, D), lambda i: (0, 0))],
        out_specs=[pl.BlockSpec((BLK, D), lambda i: (i, 0)),
                   pl.BlockSpec((BLK, D), lambda i: (i, 0))],
        out_shape=[jax.ShapeDtypeStruct((E, D), jnp.float32)] * 2,
    )(e, Eh_W, Ep_W)


_ZB = N
_NR = N + 1264


def _sc_edge(qh, kh, vh, qp, kp, vp, eh, ep, src, dst):
    mesh = plsc.VectorSubcoreMesh(core_axis_name="c", subcore_axis_name="s")
    zeros_rows = jnp.zeros((_B, D), jnp.float32)

    @functools.partial(
        pl.kernel,
        mesh=mesh,
        out_type=[jax.ShapeDtypeStruct((_NR, D), jnp.float32),
                  jax.ShapeDtypeStruct((_NR, D), jnp.float32)],
        scratch_types=[
            pltpu.VMEM((_B,), jnp.int32),
            pltpu.VMEM((_B,), jnp.int32),
            pltpu.VMEM((_B,), jnp.int32),
            pltpu.VMEM((_B, D), jnp.float32),
            pltpu.VMEM((_B, D), jnp.float32),
            pltpu.VMEM((_B, D), jnp.float32),
            pltpu.VMEM((_B, D), jnp.float32),
            pltpu.VMEM((_B, D), jnp.float32),
            pltpu.VMEM_SHARED((_NR, D), jnp.float32),
            pltpu.SemaphoreType.DMA,
            pltpu.SemaphoreType.DMA,
            pltpu.SemaphoreType.DMA,
            pltpu.SemaphoreType.DMA,
            pltpu.SemaphoreType.DMA,
            pltpu.SemaphoreType.DMA,
            pltpu.SemaphoreType.DMA,
            pltpu.SemaphoreType.DMA,
        ],
    )
    def k(qh_h, kh_h, vh_h, qp_h, kp_h, vp_h, eh_h, ep_h, src_h, dst_h,
          zrows_h,
          acch_o, accp_o,
          idx_src, idx_dst, idx8, kg, qg, vg, eg, zb8, acc_sh,
          s0, s1, s2, s3, s4, s5, s6, s7):
        c = lax.axis_index("c")
        s = lax.axis_index("s")

        _NCH = _NR // _B
        pltpu.sync_copy(zrows_h, kg)

        @pl.loop(0, (_NCH + 15) // 16)
        def zinit(jj):
            j = s + jj * 16

            @pl.when(j < _NCH)
            def _():
                pltpu.sync_copy(kg, acc_sh.at[pl.ds(j * _B, _B)])

        plsc.subcore_barrier()

        lane = lax.iota(jnp.int32, 16)

        def do_blocks(ktab, qtab, vtab, etab):
            @pl.loop(0, _NBLK)
            def blk(b):
                base = s * _EPT + b * _B
                i1 = pltpu.async_copy(src_h.at[pl.ds(base, _B)], idx_src, s0)
                i2 = pltpu.async_copy(dst_h.at[pl.ds(base, _B)], idx_dst, s1)
                i1.wait()
                i2.wait()
                for off in (0, 16):
                    dv = idx_dst[pl.ds(off, 16)]
                    idx8[pl.ds(off, 16)] = _ZB + (dv >> 3)
                c1 = pltpu.async_copy(ktab.at[idx_src], kg, s2)
                c2 = pltpu.async_copy(qtab.at[idx_dst], qg, s3)
                c3 = pltpu.async_copy(vtab.at[idx_src], vg, s4)
                c4 = pltpu.async_copy(etab.at[pl.ds(base, _B)], eg, s5)
                c1.wait()
                c2.wait()
                c3.wait()
                c4.wait()

                zvec = jnp.zeros((16,), jnp.float32)

                @pl.loop(0, _B // 16)
                def grp(g):
                    d7v = idx_dst[pl.ds(g * 16, 16)] & 7
                    for li in range(16):
                        i = g * 16 + li
                        zrow = jnp.zeros((16,), jnp.float32)
                        for hh in range(H):
                            sl = pl.ds(hh * DH, DH)
                            t = kg[i, sl] * qg[i, sl] * eg[i, sl]
                            for sh in (8, 4, 2, 1):
                                t = t + _permute16(t, lane ^ sh)
                            w = jnp.exp(jnp.clip(t, -5.0, 5.0))
                            vg[i, sl] = vg[i, sl] * w
                            zrow = jnp.where(lane == hh, w, zrow)
                        d7 = d7v[li]
                        for k8 in range(8):
                            zb8[i, pl.ds(k8 * DH, DH)] = jnp.where(
                                d7 == k8, zrow, zvec)

                a1 = pltpu.async_copy(vg, acc_sh.at[idx_dst], s6, add=True)
                a1.wait()
                a2 = pltpu.async_copy(zb8, acc_sh.at[idx8], s7, add=True)
                a2.wait()

        @pl.when(c == 0)
        def _():
            do_blocks(kh_h, qh_h, vh_h, eh_h)

        @pl.when(c == 1)
        def _():
            do_blocks(kp_h, qp_h, vp_h, ep_h)

        plsc.subcore_barrier()

        def copy_out(acc_o):
            @pl.loop(0, (_NCH + 15) // 16)
            def cout(jj):
                j = s + jj * 16

                @pl.when(j < _NCH)
                def _():
                    sl = pl.ds(j * _B, _B)
                    pltpu.sync_copy(acc_sh.at[sl], kg)
                    pltpu.sync_copy(kg, acc_o.at[sl])

        @pl.when(c == 0)
        def _():
            copy_out(acch_o)

        @pl.when(c == 1)
        def _():
            copy_out(accp_o)

    return k(qh, kh, vh, qp, kp, vp, eh, ep, src, dst, zeros_rows)


def _post_body(wvh, zh, wvp, zp, h, p, ohw, ohb, opw, opb,
               f1w, f1b, f2w, f2b, g1, b1, g2, b2, hh_o, pp_o):
    head_of_col = jax.lax.broadcasted_iota(jnp.int32, (16, D), 1) // DH
    row = jax.lax.broadcasted_iota(jnp.int32, (16, D), 0)
    sel = (head_of_col == row).astype(jnp.float32)
    zhf = jnp.dot(zh[...], sel, preferred_element_type=jnp.float32)
    zpf = jnp.dot(zp[...], sel, preferred_element_type=jnp.float32)
    h_attn = wvh[...] / (zhf + 1e-6)
    p_attn = wvp[...] / (zpf + 1e-6)
    hh = jnp.dot(h_attn, ohw[...], preferred_element_type=jnp.float32) + ohb[...]
    pp = jnp.tanh(jnp.dot(p_attn, opw[...], preferred_element_type=jnp.float32) + opb[...])
    hh = h[...] + hh
    pp_o[...] = p[...] + pp
    mu = jnp.mean(hh, axis=0, keepdims=True)
    var = jnp.mean((hh - mu) ** 2, axis=0, keepdims=True)
    hh = g1[...] * (hh - mu) / jnp.sqrt(var + 1e-5) + b1[...]
    h_in2 = hh
    hh = jnp.maximum(jnp.dot(hh, f1w[...], preferred_element_type=jnp.float32) + f1b[...], 0.0)
    hh = jnp.dot(hh, f2w[...], preferred_element_type=jnp.float32) + f2b[...]
    hh = h_in2 + hh
    mu2 = jnp.mean(hh, axis=0, keepdims=True)
    var2 = jnp.mean((hh - mu2) ** 2, axis=0, keepdims=True)
    hh_o[...] = g2[...] * (hh - mu2) / jnp.sqrt(var2 + 1e-5) + b2[...]


def _post(wvh, zh, wvp, zp, h, p, Oh_W, Oh_b, Op_W, Op_b,
          FFN1_W, FFN1_b, FFN2_W, FFN2_b, bn1_g, bn1_b, bn2_g, bn2_b):
    outs = [jax.ShapeDtypeStruct((N, D), jnp.float32)] * 2
    return pl.pallas_call(
        _post_body,
        out_shape=outs,
    )(wvh, zh, wvp, zp, h, p, Oh_W, Oh_b.reshape(1, D), Op_W, Op_b.reshape(1, D),
      FFN1_W, FFN1_b.reshape(1, 2 * D), FFN2_W, FFN2_b.reshape(1, D),
      bn1_g.reshape(1, D), bn1_b.reshape(1, D), bn2_g.reshape(1, D), bn2_b.reshape(1, D))


def kernel(h, p, e, edge_index, snorm_n, Qh_W, Kh_W, Vh_W, Eh_W, Qp_W, Kp_W,
           Ep_W, Vp_W, Oh_W, Oh_b, Op_W, Op_b, FFN1_W, FFN1_b, FFN2_W,
           FFN2_b, bn1_g, bn1_b, bn2_g, bn2_b):
    src = edge_index[0]
    dst = edge_index[1]
    qh, kh, vh, qp, kp, vp = _pre(h, p, Qh_W, Kh_W, Vh_W, Qp_W, Kp_W, Vp_W)
    ehp, epp = _eproj(e, Eh_W, Ep_W)
    acch, accp = _sc_edge(qh, kh, vh, qp, kp, vp, ehp, epp, src, dst)
    wvh = acch[:N]
    zh = acch[_ZB:_ZB + 1250].reshape(N, 16)
    wvp = accp[:N]
    zp = accp[_ZB:_ZB + 1250].reshape(N, 16)
    return _post(wvh, zh, wvp, zp, h, p, Oh_W, Oh_b, Op_W, Op_b,
                 FFN1_W, FFN1_b, FFN2_W, FFN2_b, bn1_g, bn1_b, bn2_g, bn2_b)
